# baseline (speedup 1.0000x reference)
import functools

import jax
import jax.numpy as jnp
import numpy as np
from jax import lax
from jax.experimental import pallas as pl
from jax.experimental.pallas import tpu as pltpu
from jax.experimental.pallas import tpu_sc as plsc

NC = 2
NS = 16
NW = NC * NS
L = 16

V = 10
D = 10
N_TOK = 4 * 8192
TOK_PER_W = N_TOK // NW
CHUNK = 256

_GATHER_DNUMS = lax.GatherDimensionNumbers(
    offset_dims=(), collapsed_slice_dims=(0,), start_index_map=(0,))


def _reg_gather(x, idx16):
    return lax.gather(x, idx16.reshape(L, 1), _GATHER_DNUMS, (1,),
                      mode=lax.GatherScatterMode.PROMISE_IN_BOUNDS)


def _splat(x, k):
    return _reg_gather(x, jnp.full((L,), k, jnp.int32))


def _body(ids_hbm, par_hbm, out_hbm,
          par_v, table_v, ids_v, out_a, out_b, sem, sem_a, sem_b):
    wid = lax.axis_index("s") * NC + lax.axis_index("c")
    iota16 = lax.iota(jnp.int32, L)

    cp = pltpu.async_copy(par_hbm, par_v.at[pl.ds(0, 320)], sem)
    pltpu.sync_copy(ids_hbm.at[pl.ds(wid * TOK_PER_W, TOK_PER_W)], ids_v)
    cp.wait()

    def col(base, k):
        return plsc.load_gather(par_v, [base + k + 10 * iota16])

    b1v = plsc.load_gather(par_v, [300 + iota16])
    bhv = plsc.load_gather(par_v, [310 + iota16])

    for v in range(V):
        e = plsc.load_gather(par_v, [10 * v + iota16])
        h = b1v
        for k in range(D):
            h = h + _splat(e, k) * col(100, k)
        t = bhv
        for k in range(D):
            t = t + _splat(h, k) * col(200, k)
        table_v[pl.ds(v * L, L)] = t

    bufs = (out_a, out_b)
    sems = (sem_a, sem_b)
    descs = [None, None]
    for c in range(TOK_PER_W // CHUNK):
        buf = bufs[c & 1]
        if descs[c & 1] is not None:
            descs[c & 1].wait()

        @plsc.parallel_loop(0, CHUNK // L, unroll=4)
        def blk(b):
            sid16 = ids_v[pl.ds(c * CHUNK + b * L, L)] << 4
            for j in range(L):
                idx = _reg_gather(sid16, jnp.full((L,), j, jnp.int32)) + iota16
                buf[pl.ds((b * L + j) * 128, L)] = plsc.load_gather(
                    table_v, [idx])

        descs[c & 1] = pltpu.make_async_copy(
            buf,
            out_hbm.at[pl.ds((wid * TOK_PER_W + c * CHUNK) * 128,
                             CHUNK * 128)],
            sems[c & 1],
        )
        descs[c & 1].start()
    for d in descs:
        d.wait()


@functools.partial(
    pl.kernel,
    out_type=jax.ShapeDtypeStruct((N_TOK * 128,), jnp.float32),
    mesh=plsc.VectorSubcoreMesh(core_axis_name="c", subcore_axis_name="s"),
    compiler_params=pltpu.CompilerParams(needs_layout_passes=False),
    scratch_types=[
        pltpu.VMEM((512,), jnp.float32),
        pltpu.VMEM((V * L,), jnp.float32),
        pltpu.VMEM((TOK_PER_W,), jnp.int32),
        pltpu.VMEM((CHUNK * 128,), jnp.float32),
        pltpu.VMEM((CHUNK * 128,), jnp.float32),
        pltpu.SemaphoreType.DMA,
        pltpu.SemaphoreType.DMA,
        pltpu.SemaphoreType.DMA,
    ],
)
def _sc_lookup(ids_hbm, par_hbm, out_hbm, *scratch):
    _body(ids_hbm, par_hbm, out_hbm, *scratch)


def kernel(input_ids, emb, W1, b1, Wh, bh):
    ids = input_ids.reshape(-1).astype(jnp.int32)
    par = jnp.concatenate(
        [emb.reshape(-1), W1.reshape(-1), Wh.reshape(-1), b1, bh])
    out = _sc_lookup(ids, par)
    return out.reshape(4, 8192, 128)[..., :D]


# speedup vs baseline: 1.0127x; 1.0127x over previous
import functools

import jax
import jax.numpy as jnp
import numpy as np
from jax import lax
from jax.experimental import pallas as pl
from jax.experimental.pallas import tpu as pltpu
from jax.experimental.pallas import tpu_sc as plsc

NC = 2
NS = 16
NW = NC * NS
L = 16

V = 10
D = 10
N_TOK = 4 * 8192
TOK_PER_W = N_TOK // NW
CHUNK = 256

_GATHER_DNUMS = lax.GatherDimensionNumbers(
    offset_dims=(), collapsed_slice_dims=(0,), start_index_map=(0,))


def _reg_gather(x, idx16):
    return lax.gather(x, idx16.reshape(L, 1), _GATHER_DNUMS, (1,),
                      mode=lax.GatherScatterMode.PROMISE_IN_BOUNDS)


def _splat(x, k):
    return _reg_gather(x, jnp.full((L,), k, jnp.int32))


def _body(ids_hbm, par_hbm, out_hbm,
          par_v, table_v, ids_v, out_a, out_b, sem, sem_a, sem_b):
    wid = lax.axis_index("s") * NC + lax.axis_index("c")
    iota16 = lax.iota(jnp.int32, L)

    cp = pltpu.async_copy(par_hbm, par_v.at[pl.ds(0, 320)], sem)
    pltpu.sync_copy(ids_hbm.at[pl.ds(wid * TOK_PER_W, TOK_PER_W)], ids_v)
    cp.wait()

    def col(base, k):
        return plsc.load_gather(par_v, [base + k + 10 * iota16])

    b1v = plsc.load_gather(par_v, [300 + iota16])
    bhv = plsc.load_gather(par_v, [310 + iota16])

    for v in range(V):
        e = plsc.load_gather(par_v, [10 * v + iota16])
        h = b1v
        for k in range(D):
            h = h + _splat(e, k) * col(100, k)
        t = bhv
        for k in range(D):
            t = t + _splat(h, k) * col(200, k)
        table_v[pl.ds(v * L, L)] = t

    bufs = (out_a, out_b)
    sems = (sem_a, sem_b)
    descs = [None, None]
    for c in range(TOK_PER_W // CHUNK):
        buf = bufs[c & 1]
        if descs[c & 1] is not None:
            descs[c & 1].wait()

        @plsc.parallel_loop(0, CHUNK // L, unroll=2)
        def blk(b):
            sid16 = ids_v[pl.ds(c * CHUNK + b * L, L)] << 4
            for j in range(L):
                idx = _reg_gather(sid16, jnp.full((L,), j, jnp.int32)) + iota16
                buf[pl.ds((b * L + j) * 128, L)] = plsc.load_gather(
                    table_v, [idx])

        descs[c & 1] = pltpu.make_async_copy(
            buf,
            out_hbm.at[pl.ds((wid * TOK_PER_W + c * CHUNK) * 128,
                             CHUNK * 128)],
            sems[c & 1],
        )
        descs[c & 1].start()
    for d in descs:
        d.wait()


@functools.partial(
    pl.kernel,
    out_type=jax.ShapeDtypeStruct((N_TOK * 128,), jnp.float32),
    mesh=plsc.VectorSubcoreMesh(core_axis_name="c", subcore_axis_name="s"),
    compiler_params=pltpu.CompilerParams(
        needs_layout_passes=False, skip_device_barrier=True),
    scratch_types=[
        pltpu.VMEM((512,), jnp.float32),
        pltpu.VMEM((V * L,), jnp.float32),
        pltpu.VMEM((TOK_PER_W,), jnp.int32),
        pltpu.VMEM((CHUNK * 128,), jnp.float32),
        pltpu.VMEM((CHUNK * 128,), jnp.float32),
        pltpu.SemaphoreType.DMA,
        pltpu.SemaphoreType.DMA,
        pltpu.SemaphoreType.DMA,
    ],
)
def _sc_lookup(ids_hbm, par_hbm, out_hbm, *scratch):
    _body(ids_hbm, par_hbm, out_hbm, *scratch)


def kernel(input_ids, emb, W1, b1, Wh, bh):
    ids = input_ids.reshape(-1).astype(jnp.int32)
    par = jnp.concatenate(
        [emb.reshape(-1), W1.reshape(-1), Wh.reshape(-1), b1, bh])
    out = _sc_lookup(ids, par)
    return out.reshape(4, 8192, 128)[..., :D]
